# Initial kernel scaffold; baseline (speedup 1.0000x reference)
import jax
import jax.numpy as jnp
from jax.experimental import pallas as pl


def kernel(x, edge_index, batch, W1, b1, W2, b2, Wlin, blin):
    raise NotImplementedError("write your pallas kernel here")


# speedup vs baseline: 12.8974x; 12.8974x over previous
import functools

import jax
import jax.numpy as jnp
from jax import lax
from jax.experimental import pallas as pl
from jax.experimental.pallas import tpu as pltpu
from jax.experimental.pallas import tpu_sc as plsc

N_NODE = 10000
N_EDGE = 320000
FEAT = 128
N_GRAPH = 64
N_CLS = 10

N_TILES = 32
E_PER_TILE = N_EDGE // N_TILES
CHUNK = 80
N_CHUNK = E_PER_TILE // CHUNK
N_PAD = 10240
R_PER_TILE = N_PAD // 16

_MESH = plsc.VectorSubcoreMesh(core_axis_name="c", subcore_axis_name="s")


def _deg_body(dst_hbm, ones_hbm, zeros_hbm, out_hbm, didx, ones_v, acc, sem):
    cid = lax.axis_index("c")
    sid = lax.axis_index("s")
    pltpu.sync_copy(zeros_hbm, acc.at[pl.ds(sid * R_PER_TILE, R_PER_TILE)])
    pltpu.sync_copy(ones_hbm, ones_v)
    plsc.subcore_barrier()
    tile_base = (cid * 16 + sid) * E_PER_TILE

    def body(it, carry):
        base = pl.multiple_of(tile_base + it * CHUNK, 8)
        pltpu.sync_copy(dst_hbm.at[pl.ds(base, CHUNK)], didx)
        pltpu.sync_copy(ones_v, acc.at[didx], add=True)
        return carry

    lax.fori_loop(0, N_CHUNK, body, 0)
    plsc.subcore_barrier()
    pltpu.sync_copy(acc.at[pl.ds(sid * R_PER_TILE, R_PER_TILE)],
                    out_hbm.at[cid, pl.ds(sid * R_PER_TILE, R_PER_TILE)])


@functools.partial(jax.jit, donate_argnums=())
def _sc_degree(dst, ones16, zeros16):
    return pl.kernel(
        _deg_body,
        out_type=jax.ShapeDtypeStruct((2, N_PAD, 16), jnp.float32),
        mesh=_MESH,
        scratch_types=[
            pltpu.VMEM((CHUNK,), jnp.int32),
            pltpu.VMEM((CHUNK, 16), jnp.float32),
            pltpu.VMEM_SHARED((N_PAD, 16), jnp.float32),
            pltpu.SemaphoreType.DMA,
        ],
    )(dst, ones16, zeros16)


def _scat_body(g_hbm, src_hbm, dst_hbm, zeros_hbm, out_hbm,
               sidx, didx, rows, acc, sem):
    cid = lax.axis_index("c")
    sid = lax.axis_index("s")
    pltpu.sync_copy(zeros_hbm, acc.at[pl.ds(sid * R_PER_TILE, R_PER_TILE)])
    plsc.subcore_barrier()
    tile_base = (cid * 16 + sid) * E_PER_TILE

    def body(it, carry):
        base = pl.multiple_of(tile_base + it * CHUNK, 8)
        pltpu.sync_copy(src_hbm.at[pl.ds(base, CHUNK)], sidx)
        pltpu.sync_copy(dst_hbm.at[pl.ds(base, CHUNK)], didx)
        pltpu.async_copy(g_hbm.at[sidx], rows, sem).wait()
        pltpu.sync_copy(rows, acc.at[didx], add=True)
        return carry

    lax.fori_loop(0, N_CHUNK, body, 0)
    plsc.subcore_barrier()
    pltpu.sync_copy(acc.at[pl.ds(sid * R_PER_TILE, R_PER_TILE)],
                    out_hbm.at[cid, pl.ds(sid * R_PER_TILE, R_PER_TILE)])


def _sc_scatter(g, src, dst, zeros128):
    return pl.kernel(
        _scat_body,
        out_type=jax.ShapeDtypeStruct((2, N_PAD, FEAT), jnp.float32),
        mesh=_MESH,
        scratch_types=[
            pltpu.VMEM((CHUNK,), jnp.int32),
            pltpu.VMEM((CHUNK,), jnp.int32),
            pltpu.VMEM((CHUNK, FEAT), jnp.float32),
            pltpu.VMEM_SHARED((N_PAD, FEAT), jnp.float32),
            pltpu.SemaphoreType.DMA,
        ],
    )(g, src, dst, zeros128)


_BLK = 1000
_GRID = N_NODE // _BLK


def _dinv_from(degp_ref):
    deg = degp_ref[0, :, 0:1] + degp_ref[1, :, 0:1] + 1.0
    return lax.rsqrt(deg)


def _tc1_body(degp_ref, x_ref, w1_ref, g1_ref):
    dinv = _dinv_from(degp_ref)
    h = jnp.dot(x_ref[...], w1_ref[...], preferred_element_type=jnp.float32)
    g1_ref[...] = dinv * h


def _tc1(degp, x, W1):
    return pl.pallas_call(
        _tc1_body,
        grid=(_GRID,),
        in_specs=[
            pl.BlockSpec((2, _BLK, 16), lambda i: (0, i, 0)),
            pl.BlockSpec((_BLK, FEAT), lambda i: (i, 0)),
            pl.BlockSpec((FEAT, FEAT), lambda i: (0, 0)),
        ],
        out_specs=pl.BlockSpec((_BLK, FEAT), lambda i: (i, 0)),
        out_shape=jax.ShapeDtypeStruct((N_NODE, FEAT), jnp.float32),
    )(degp, x, W1)


def _tc2_body(p_ref, g1_ref, degp_ref, w2_ref, b1_ref, g2_ref):
    dinv = _dinv_from(degp_ref)
    h1 = dinv * (p_ref[0] + p_ref[1] + g1_ref[...]) + b1_ref[...]
    h1 = jnp.maximum(h1, 0.0)
    g2_ref[...] = dinv * jnp.dot(h1, w2_ref[...],
                                 preferred_element_type=jnp.float32)


def _tc2(P1, g1, degp, W2, b1):
    return pl.pallas_call(
        _tc2_body,
        grid=(_GRID,),
        in_specs=[
            pl.BlockSpec((2, _BLK, FEAT), lambda i: (0, i, 0)),
            pl.BlockSpec((_BLK, FEAT), lambda i: (i, 0)),
            pl.BlockSpec((2, _BLK, 16), lambda i: (0, i, 0)),
            pl.BlockSpec((FEAT, FEAT), lambda i: (0, 0)),
            pl.BlockSpec((1, FEAT), lambda i: (0, 0)),
        ],
        out_specs=pl.BlockSpec((_BLK, FEAT), lambda i: (i, 0)),
        out_shape=jax.ShapeDtypeStruct((N_NODE, FEAT), jnp.float32),
    )(P1, g1, degp, W2, b1)


def _tc3_body(p_ref, g2_ref, degp_ref, b2_ref, batch_ref, wlin_ref, blin_ref,
              out_ref, pooled_scr, cnt_scr):
    i = pl.program_id(0)

    @pl.when(i == 0)
    def _():
        pooled_scr[...] = jnp.zeros_like(pooled_scr)
        cnt_scr[...] = jnp.zeros_like(cnt_scr)

    dinv = _dinv_from(degp_ref)
    h2 = dinv * (p_ref[0] + p_ref[1] + g2_ref[...]) + b2_ref[...]
    h2 = jnp.maximum(h2, 0.0)
    cols = lax.broadcasted_iota(jnp.int32, (_BLK, N_GRAPH), 1)
    oh = (batch_ref[...] == cols).astype(jnp.float32)
    dims = (((0,), (0,)), ((), ()))
    pooled_scr[...] += lax.dot_general(oh, h2, dims,
                                       preferred_element_type=jnp.float32)
    cnt_scr[...] += lax.dot_general(oh, jnp.ones_like(h2), dims,
                                    preferred_element_type=jnp.float32)
    pooled = pooled_scr[...] / jnp.maximum(cnt_scr[...], 1.0)
    out_ref[...] = jnp.dot(pooled, wlin_ref[...],
                           preferred_element_type=jnp.float32) + blin_ref[...]


def _tc3(P2, g2, degp, b2, batch, Wlin, blin):
    return pl.pallas_call(
        _tc3_body,
        grid=(_GRID,),
        in_specs=[
            pl.BlockSpec((2, _BLK, FEAT), lambda i: (0, i, 0)),
            pl.BlockSpec((_BLK, FEAT), lambda i: (i, 0)),
            pl.BlockSpec((2, _BLK, 16), lambda i: (0, i, 0)),
            pl.BlockSpec((1, FEAT), lambda i: (0, 0)),
            pl.BlockSpec((_BLK, 1), lambda i: (i, 0)),
            pl.BlockSpec((FEAT, N_CLS), lambda i: (0, 0)),
            pl.BlockSpec((1, N_CLS), lambda i: (0, 0)),
        ],
        out_specs=pl.BlockSpec((N_GRAPH, N_CLS), lambda i: (0, 0)),
        out_shape=jax.ShapeDtypeStruct((N_GRAPH, N_CLS), jnp.float32),
        scratch_shapes=[
            pltpu.VMEM((N_GRAPH, FEAT), jnp.float32),
            pltpu.VMEM((N_GRAPH, FEAT), jnp.float32),
        ],
    )(P2, g2, degp, b2, batch, Wlin, blin)


def kernel(x, edge_index, batch, W1, b1, W2, b2, Wlin, blin):
    src = edge_index[0].astype(jnp.int32)
    dst = edge_index[1].astype(jnp.int32)
    batch_i = batch.astype(jnp.int32).reshape(N_NODE, 1)
    zeros128 = jnp.zeros((R_PER_TILE, FEAT), jnp.float32)
    zeros16 = jnp.zeros((R_PER_TILE, 16), jnp.float32)
    ones16 = jnp.ones((CHUNK, 16), jnp.float32)

    degp = _sc_degree(dst, ones16, zeros16)
    g1 = _tc1(degp, x, W1)
    P1 = _sc_scatter(g1, src, dst, zeros128)
    g2 = _tc2(P1, g1, degp, W2, b1.reshape(1, FEAT))
    P2 = _sc_scatter(g2, src, dst, zeros128)
    return _tc3(P2, g2, degp, b2.reshape(1, FEAT), batch_i,
                Wlin, blin.reshape(1, N_CLS))
